# baseline (speedup 1.0000x reference)
import jax
import jax.numpy as jnp
from jax.experimental import pallas as pl
from jax.experimental.pallas import tpu as pltpu

_N = 4096
_D = 128
_H = 64
_Z = 16
_CAUSAL = 8
_E = 131072
_M = 2
_SIZE_COEF = 1e-4
_KL_COEF = 1.0
_VGAE_COEF = 1.0

_RT = 128
_G = _N // _RT


def _sweep_body(z_ref, zt_ref, ct_ref, mu_ref, lv_ref, org_ref, out_ref):
    i = pl.program_id(0)
    z = z_ref[...]
    s = jnp.dot(z, zt_ref[...], preferred_element_type=jnp.float32)
    r = jax.nn.sigmoid(s)
    log_neg = jnp.log(1.0 - r + 1e-8)
    bce_all = jnp.sum(log_neg)
    rows = jax.lax.broadcasted_iota(jnp.int32, (_RT, _N), 0) + i * _RT
    cols = jax.lax.broadcasted_iota(jnp.int32, (_RT, _N), 1)
    on = jnp.logical_or(org_ref[...].astype(jnp.float32) > 0.0, rows == cols)
    corr = jnp.sum(jnp.where(on, jnp.log(r + 1e-8) - log_neg, 0.0))
    c = z[:, :_CAUSAL]
    sc = jnp.dot(c, ct_ref[...], preferred_element_type=jnp.float32)
    size_sum = jnp.sum(jax.nn.sigmoid(sc))
    muv = mu_ref[...]
    lvv = lv_ref[...]
    klz_part = jnp.sum(1.0 + lvv - muv * muv - jnp.exp(lvv))
    row = jnp.concatenate([
        jnp.reshape(bce_all + corr, (1,)),
        jnp.reshape(size_sum, (1,)),
        jnp.reshape(klz_part, (1,)),
        jnp.zeros((125,), jnp.float32),
    ])[None, :]
    out_ref[pl.ds(i, 1), :] = row


def _decoder_losses(all_z, mu, logvar, org_mask):
    zt = all_z.T
    ct = all_z[:, :_CAUSAL].T
    out = pl.pallas_call(
        _sweep_body,
        grid=(_G,),
        in_specs=[
            pl.BlockSpec((_RT, _Z), lambda i: (i, 0)),
            pl.BlockSpec((_Z, _N), lambda i: (0, 0)),
            pl.BlockSpec((_CAUSAL, _N), lambda i: (0, 0)),
            pl.BlockSpec((_RT, _Z), lambda i: (i, 0)),
            pl.BlockSpec((_RT, _Z), lambda i: (i, 0)),
            pl.BlockSpec((_RT, _N), lambda i: (i, 0)),
        ],
        out_specs=pl.BlockSpec((_G, 128), lambda i: (0, 0)),
        out_shape=jax.ShapeDtypeStruct((_G, 128), jnp.float32),
    )(all_z, zt, ct, mu, logvar, org_mask)
    sums = jnp.sum(out, axis=0)
    bce = -sums[0] / (_N * _N)
    size_loss = _SIZE_COEF * sums[1] / (_N * _N)
    klz = (-0.5 / _N) * sums[2]
    return bce, size_loss, klz


def kernel(x, edge_index, eps, W1, W_mu, W_lv, Wc1, Wc2, Wout):
    src = edge_index[0]
    dst = edge_index[1]

    x_aug = jnp.concatenate([x, jnp.ones((_N, 1), jnp.float32)], axis=1)
    s_xa = jax.ops.segment_sum(jnp.take(x_aug, src, axis=0), dst, num_segments=_N)
    s_x = s_xa[:, :_D]
    deg = jnp.clip(s_xa[:, _D:], 1.0)

    hx1 = x @ Wc1
    h1o = jax.nn.relu((s_x @ Wc1) / deg + hx1)
    h = jax.nn.relu((s_x @ W1) / deg + x @ W1)
    s_cat = jax.ops.segment_sum(
        jnp.take(jnp.concatenate([h1o, h], axis=1), src, axis=0), dst,
        num_segments=_N)
    s1o = s_cat[:, :_H]
    s_h = s_cat[:, _H:]
    h2o = jax.nn.relu((s1o @ Wc2) / deg + h1o @ Wc2)
    orig_logits = jnp.mean(h2o, axis=0, keepdims=True) @ Wout
    mu = (s_h @ W_mu) / deg + h @ W_mu
    logvar = (s_h @ W_lv) / deg + h @ W_lv
    all_z = mu + jnp.exp(0.5 * logvar) * eps
    caul_z = all_z[:, :_CAUSAL]

    c_src = jnp.take(caul_z, src, axis=0)
    c_dst = jnp.take(caul_z, dst, axis=0)
    edge_attn = jax.nn.sigmoid(jnp.sum(c_src * c_dst, axis=1, keepdims=True))

    s_xw = jax.ops.segment_sum(jnp.take(x, src, axis=0) * edge_attn, dst,
                               num_segments=_N)
    h1m = jax.nn.relu((s_xw @ Wc1) / deg + hx1)
    s_m = jax.ops.segment_sum(jnp.take(h1m, src, axis=0) * edge_attn, dst,
                              num_segments=_N)
    h2m = jax.nn.relu((s_m @ Wc2) / deg + h1m @ Wc2)
    masked_logits = jnp.mean(h2m, axis=0, keepdims=True) @ Wout

    org_mask = (jnp.zeros((_N, _N), jnp.int8)
                .at[src, dst].set(1)
                .at[dst, src].set(1))

    bce, size_loss, klz = _decoder_losses(all_z, mu, logvar, org_mask)

    logp = jax.nn.log_softmax(masked_logits, axis=1)
    p = jax.nn.softmax(orig_logits, axis=1)
    kl_loss = _KL_COEF * jnp.sum(p * (jnp.log(p + 1e-12) - logp)) / masked_logits.shape[0]
    vgae_loss = _VGAE_COEF * (bce + klz)
    loss = size_loss + kl_loss + vgae_loss
    return (loss, orig_logits, edge_attn.reshape(-1))


# speedup vs baseline: 1.0272x; 1.0272x over previous
import jax
import jax.numpy as jnp
from jax.experimental import pallas as pl
from jax.experimental.pallas import tpu as pltpu

_N = 4096
_D = 128
_H = 64
_Z = 16
_CAUSAL = 8
_E = 131072
_M = 2
_SIZE_COEF = 1e-4
_KL_COEF = 1.0
_VGAE_COEF = 1.0

_RT = 128
_G = _N // _RT


def _sweep_body(z_ref, zt_ref, ct_ref, mu_ref, lv_ref, org_ref, out_ref):
    i = pl.program_id(0)
    z = z_ref[...]
    s = jnp.dot(z, zt_ref[...], preferred_element_type=jnp.float32)
    r = jax.nn.sigmoid(s)
    log_neg = jnp.log(1.0 - r + 1e-8)
    bce_all = jnp.sum(log_neg)
    rows = jax.lax.broadcasted_iota(jnp.int32, (_RT, _N), 0) + i * _RT
    cols = jax.lax.broadcasted_iota(jnp.int32, (_RT, _N), 1)
    on = jnp.logical_or(org_ref[...] > 0.0, rows == cols)
    corr = jnp.sum(jnp.where(on, jnp.log(r + 1e-8) - log_neg, 0.0))
    c = z[:, :_CAUSAL]
    sc = jnp.dot(c, ct_ref[...], preferred_element_type=jnp.float32)
    size_sum = jnp.sum(jax.nn.sigmoid(sc))
    muv = mu_ref[...]
    lvv = lv_ref[...]
    klz_part = jnp.sum(1.0 + lvv - muv * muv - jnp.exp(lvv))
    row = jnp.concatenate([
        jnp.reshape(bce_all + corr, (1,)),
        jnp.reshape(size_sum, (1,)),
        jnp.reshape(klz_part, (1,)),
        jnp.zeros((125,), jnp.float32),
    ])[None, :]
    out_ref[pl.ds(i, 1), :] = row


def _decoder_losses(all_z, mu, logvar, org_mask):
    zt = all_z.T
    ct = all_z[:, :_CAUSAL].T
    out = pl.pallas_call(
        _sweep_body,
        grid=(_G,),
        in_specs=[
            pl.BlockSpec((_RT, _Z), lambda i: (i, 0)),
            pl.BlockSpec((_Z, _N), lambda i: (0, 0)),
            pl.BlockSpec((_CAUSAL, _N), lambda i: (0, 0)),
            pl.BlockSpec((_RT, _Z), lambda i: (i, 0)),
            pl.BlockSpec((_RT, _Z), lambda i: (i, 0)),
            pl.BlockSpec((_RT, _N), lambda i: (i, 0)),
        ],
        out_specs=pl.BlockSpec((_G, 128), lambda i: (0, 0)),
        out_shape=jax.ShapeDtypeStruct((_G, 128), jnp.float32),
    )(all_z, zt, ct, mu, logvar, org_mask)
    sums = jnp.sum(out, axis=0)
    bce = -sums[0] / (_N * _N)
    size_loss = _SIZE_COEF * sums[1] / (_N * _N)
    klz = (-0.5 / _N) * sums[2]
    return bce, size_loss, klz


def kernel(x, edge_index, eps, W1, W_mu, W_lv, Wc1, Wc2, Wout):
    src = edge_index[0]
    dst = edge_index[1]

    x_aug = jnp.concatenate([x, jnp.ones((_N, 1), jnp.float32)], axis=1)
    s_xa = jax.ops.segment_sum(jnp.take(x_aug, src, axis=0), dst, num_segments=_N)
    s_x = s_xa[:, :_D]
    deg = jnp.clip(s_xa[:, _D:], 1.0)

    hx1 = x @ Wc1
    h1o = jax.nn.relu((s_x @ Wc1) / deg + hx1)
    h = jax.nn.relu((s_x @ W1) / deg + x @ W1)
    s_cat = jax.ops.segment_sum(
        jnp.take(jnp.concatenate([h1o, h], axis=1), src, axis=0), dst,
        num_segments=_N)
    s1o = s_cat[:, :_H]
    s_h = s_cat[:, _H:]
    h2o = jax.nn.relu((s1o @ Wc2) / deg + h1o @ Wc2)
    orig_logits = jnp.mean(h2o, axis=0, keepdims=True) @ Wout
    mu = (s_h @ W_mu) / deg + h @ W_mu
    logvar = (s_h @ W_lv) / deg + h @ W_lv
    all_z = mu + jnp.exp(0.5 * logvar) * eps
    caul_z = all_z[:, :_CAUSAL]

    c_src = jnp.take(caul_z, src, axis=0)
    c_dst = jnp.take(caul_z, dst, axis=0)
    edge_attn = jax.nn.sigmoid(jnp.sum(c_src * c_dst, axis=1, keepdims=True))

    s_xw = jax.ops.segment_sum(jnp.take(x, src, axis=0) * edge_attn, dst,
                               num_segments=_N)
    h1m = jax.nn.relu((s_xw @ Wc1) / deg + hx1)
    s_m = jax.ops.segment_sum(jnp.take(h1m, src, axis=0) * edge_attn, dst,
                              num_segments=_N)
    h2m = jax.nn.relu((s_m @ Wc2) / deg + h1m @ Wc2)
    masked_logits = jnp.mean(h2m, axis=0, keepdims=True) @ Wout

    org_mask = (jnp.zeros((_N, _N), jnp.float32)
                .at[src, dst].set(1.0)
                .at[dst, src].set(1.0))

    bce, size_loss, klz = _decoder_losses(all_z, mu, logvar, org_mask)

    logp = jax.nn.log_softmax(masked_logits, axis=1)
    p = jax.nn.softmax(orig_logits, axis=1)
    kl_loss = _KL_COEF * jnp.sum(p * (jnp.log(p + 1e-12) - logp)) / masked_logits.shape[0]
    vgae_loss = _VGAE_COEF * (bce + klz)
    loss = size_loss + kl_loss + vgae_loss
    return (loss, orig_logits, edge_attn.reshape(-1))


# speedup vs baseline: 1.0288x; 1.0015x over previous
import jax
import jax.numpy as jnp
from jax.experimental import pallas as pl
from jax.experimental.pallas import tpu as pltpu

_N = 4096
_D = 128
_H = 64
_Z = 16
_CAUSAL = 8
_E = 131072
_M = 2
_SIZE_COEF = 1e-4
_KL_COEF = 1.0
_VGAE_COEF = 1.0

_RT = 128
_G = _N // _RT


def _sweep_body(z_ref, zt_ref, ct_ref, mu_ref, lv_ref, org_ref, out_ref):
    i = pl.program_id(0)
    z = z_ref[...]
    s = jnp.dot(z, zt_ref[...], preferred_element_type=jnp.float32)
    r = jax.nn.sigmoid(s)
    log_neg = jnp.log(1.0 - r + 1e-8)
    bce_all = jnp.sum(log_neg)
    rows = jax.lax.broadcasted_iota(jnp.int32, (_RT, _N), 0) + i * _RT
    cols = jax.lax.broadcasted_iota(jnp.int32, (_RT, _N), 1)
    on = jnp.logical_or(org_ref[...] > 0.0, rows == cols)
    corr = jnp.sum(jnp.where(on, jnp.log(r + 1e-8) - log_neg, 0.0))
    c = z[:, :_CAUSAL]
    sc = jnp.dot(c, ct_ref[...], preferred_element_type=jnp.float32)
    size_sum = jnp.sum(jax.nn.sigmoid(sc))
    muv = mu_ref[...]
    lvv = lv_ref[...]
    klz_part = jnp.sum(1.0 + lvv - muv * muv - jnp.exp(lvv))
    row = jnp.concatenate([
        jnp.reshape(bce_all + corr, (1,)),
        jnp.reshape(size_sum, (1,)),
        jnp.reshape(klz_part, (1,)),
        jnp.zeros((125,), jnp.float32),
    ])[None, :]
    out_ref[pl.ds(i, 1), :] = row


def _decoder_losses(all_z, mu, logvar, org_mask):
    zt = all_z.T
    ct = all_z[:, :_CAUSAL].T
    out = pl.pallas_call(
        _sweep_body,
        grid=(_G,),
        in_specs=[
            pl.BlockSpec((_RT, _Z), lambda i: (i, 0)),
            pl.BlockSpec((_Z, _N), lambda i: (0, 0)),
            pl.BlockSpec((_CAUSAL, _N), lambda i: (0, 0)),
            pl.BlockSpec((_RT, _Z), lambda i: (i, 0)),
            pl.BlockSpec((_RT, _Z), lambda i: (i, 0)),
            pl.BlockSpec((_RT, _N), lambda i: (i, 0)),
        ],
        out_specs=pl.BlockSpec((_G, 128), lambda i: (0, 0)),
        out_shape=jax.ShapeDtypeStruct((_G, 128), jnp.float32),
    )(all_z, zt, ct, mu, logvar, org_mask)
    sums = jnp.sum(out, axis=0)
    bce = -sums[0] / (_N * _N)
    size_loss = _SIZE_COEF * sums[1] / (_N * _N)
    klz = (-0.5 / _N) * sums[2]
    return bce, size_loss, klz


def kernel(x, edge_index, eps, W1, W_mu, W_lv, Wc1, Wc2, Wout):
    src = edge_index[0]
    dst = edge_index[1]

    x_aug = jnp.concatenate([x, jnp.ones((_N, 1), jnp.float32)], axis=1)
    g_xa = jnp.take(x_aug, src, axis=0)
    s_xa = jax.ops.segment_sum(g_xa, dst, num_segments=_N)
    s_x = s_xa[:, :_D]
    deg = jnp.clip(s_xa[:, _D:], 1.0)

    hx1 = x @ Wc1
    h1o = jax.nn.relu((s_x @ Wc1) / deg + hx1)
    h = jax.nn.relu((s_x @ W1) / deg + x @ W1)
    s_cat = jax.ops.segment_sum(
        jnp.take(jnp.concatenate([h1o, h], axis=1), src, axis=0), dst,
        num_segments=_N)
    s1o = s_cat[:, :_H]
    s_h = s_cat[:, _H:]
    h2o = jax.nn.relu((s1o @ Wc2) / deg + h1o @ Wc2)
    orig_logits = jnp.mean(h2o, axis=0, keepdims=True) @ Wout
    mu = (s_h @ W_mu) / deg + h @ W_mu
    logvar = (s_h @ W_lv) / deg + h @ W_lv
    all_z = mu + jnp.exp(0.5 * logvar) * eps
    caul_z = all_z[:, :_CAUSAL]

    c_both = jnp.take(caul_z, jnp.concatenate([src, dst]), axis=0)
    edge_attn = jax.nn.sigmoid(
        jnp.sum(c_both[:_E] * c_both[_E:], axis=1, keepdims=True))

    s_xw = jax.ops.segment_sum(g_xa[:, :_D] * edge_attn, dst,
                               num_segments=_N)
    h1m = jax.nn.relu((s_xw @ Wc1) / deg + hx1)
    s_m = jax.ops.segment_sum(jnp.take(h1m, src, axis=0) * edge_attn, dst,
                              num_segments=_N)
    h2m = jax.nn.relu((s_m @ Wc2) / deg + h1m @ Wc2)
    masked_logits = jnp.mean(h2m, axis=0, keepdims=True) @ Wout

    org_mask = (jnp.zeros((_N, _N), jnp.float32)
                .at[src, dst].set(1.0)
                .at[dst, src].set(1.0))

    bce, size_loss, klz = _decoder_losses(all_z, mu, logvar, org_mask)

    logp = jax.nn.log_softmax(masked_logits, axis=1)
    p = jax.nn.softmax(orig_logits, axis=1)
    kl_loss = _KL_COEF * jnp.sum(p * (jnp.log(p + 1e-12) - logp)) / masked_logits.shape[0]
    vgae_loss = _VGAE_COEF * (bce + klz)
    loss = size_loss + kl_loss + vgae_loss
    return (loss, orig_logits, edge_attn.reshape(-1))


# speedup vs baseline: 1.1906x; 1.1574x over previous
import jax
import jax.numpy as jnp
from jax.experimental import pallas as pl
from jax.experimental.pallas import tpu as pltpu

_N = 4096
_D = 128
_H = 64
_Z = 16
_CAUSAL = 8
_E = 131072
_M = 2
_SIZE_COEF = 1e-4
_KL_COEF = 1.0
_VGAE_COEF = 1.0

_RT = 128
_G = _N // _RT


def _sweep_body(z_ref, zt_ref, ct_ref, mu_ref, lv_ref, org_ref, out_ref):
    i = pl.program_id(0)
    z = z_ref[...]
    s = jnp.dot(z, zt_ref[...], preferred_element_type=jnp.float32)
    r = jax.nn.sigmoid(s)
    log_neg = jnp.log(1.0 - r + 1e-8)
    bce_all = jnp.sum(log_neg)
    rows = jax.lax.broadcasted_iota(jnp.int32, (_RT, _N), 0) + i * _RT
    cols = jax.lax.broadcasted_iota(jnp.int32, (_RT, _N), 1)
    on = jnp.logical_or(org_ref[...] > 0.0, rows == cols)
    corr = jnp.sum(jnp.where(on, jnp.log(r + 1e-8) - log_neg, 0.0))
    c = z[:, :_CAUSAL]
    sc = jnp.dot(c, ct_ref[...], preferred_element_type=jnp.float32)
    size_sum = jnp.sum(jax.nn.sigmoid(sc))
    muv = mu_ref[...]
    lvv = lv_ref[...]
    klz_part = jnp.sum(1.0 + lvv - muv * muv - jnp.exp(lvv))
    row = jnp.concatenate([
        jnp.reshape(bce_all + corr, (1,)),
        jnp.reshape(size_sum, (1,)),
        jnp.reshape(klz_part, (1,)),
        jnp.zeros((125,), jnp.float32),
    ])[None, :]
    out_ref[pl.ds(i, 1), :] = row


def _decoder_losses(all_z, mu, logvar, org_mask):
    zt = all_z.T
    ct = all_z[:, :_CAUSAL].T
    out = pl.pallas_call(
        _sweep_body,
        grid=(_G,),
        in_specs=[
            pl.BlockSpec((_RT, _Z), lambda i: (i, 0)),
            pl.BlockSpec((_Z, _N), lambda i: (0, 0)),
            pl.BlockSpec((_CAUSAL, _N), lambda i: (0, 0)),
            pl.BlockSpec((_RT, _Z), lambda i: (i, 0)),
            pl.BlockSpec((_RT, _Z), lambda i: (i, 0)),
            pl.BlockSpec((_RT, _N), lambda i: (i, 0)),
        ],
        out_specs=pl.BlockSpec((_G, 128), lambda i: (0, 0)),
        out_shape=jax.ShapeDtypeStruct((_G, 128), jnp.float32),
    )(all_z, zt, ct, mu, logvar, org_mask)
    sums = jnp.sum(out, axis=0)
    bce = -sums[0] / (_N * _N)
    size_loss = _SIZE_COEF * sums[1] / (_N * _N)
    klz = (-0.5 / _N) * sums[2]
    return bce, size_loss, klz


def kernel(x, edge_index, eps, W1, W_mu, W_lv, Wc1, Wc2, Wout):
    src = edge_index[0]
    dst = edge_index[1]

    x_aug = jnp.concatenate([x, jnp.ones((_N, 1), jnp.float32)], axis=1)
    g_xa = jnp.take(x_aug, src, axis=0)
    s_xa = jax.ops.segment_sum(g_xa, dst, num_segments=_N)
    s_x = s_xa[:, :_D]
    deg = jnp.clip(s_xa[:, _D:], 1.0)

    hx1 = x @ Wc1
    h1o = jax.nn.relu((s_x @ Wc1) / deg + hx1)
    h = jax.nn.relu((s_x @ W1) / deg + x @ W1)
    s_cat = jax.ops.segment_sum(
        jnp.take(jnp.concatenate([h1o, h], axis=1), src, axis=0), dst,
        num_segments=_N)
    s1o = s_cat[:, :_H]
    s_h = s_cat[:, _H:]
    h2o = jax.nn.relu((s1o @ Wc2) / deg + h1o @ Wc2)
    orig_logits = jnp.mean(h2o, axis=0, keepdims=True) @ Wout
    mu = (s_h @ W_mu) / deg + h @ W_mu
    logvar = (s_h @ W_lv) / deg + h @ W_lv
    all_z = mu + jnp.exp(0.5 * logvar) * eps
    caul_z = all_z[:, :_CAUSAL]

    c_both = jnp.take(caul_z, jnp.concatenate([src, dst]), axis=0)
    edge_attn = jax.nn.sigmoid(
        jnp.sum(c_both[:_E] * c_both[_E:], axis=1, keepdims=True))

    s_xw = jax.ops.segment_sum(g_xa[:, :_D] * edge_attn, dst,
                               num_segments=_N)
    h1m = jax.nn.relu((s_xw @ Wc1) / deg + hx1)
    s_m = jax.ops.segment_sum(jnp.take(h1m, src, axis=0) * edge_attn, dst,
                              num_segments=_N)
    h2m = jax.nn.relu((s_m @ Wc2) / deg + h1m @ Wc2)
    masked_logits = jnp.mean(h2m, axis=0, keepdims=True) @ Wout

    org_mask = jnp.zeros((_N, _N), jnp.float32).at[
        jnp.concatenate([src, dst]), jnp.concatenate([dst, src])].add(1.0)

    bce, size_loss, klz = _decoder_losses(all_z, mu, logvar, org_mask)

    logp = jax.nn.log_softmax(masked_logits, axis=1)
    p = jax.nn.softmax(orig_logits, axis=1)
    kl_loss = _KL_COEF * jnp.sum(p * (jnp.log(p + 1e-12) - logp)) / masked_logits.shape[0]
    vgae_loss = _VGAE_COEF * (bce + klz)
    loss = size_loss + kl_loss + vgae_loss
    return (loss, orig_logits, edge_attn.reshape(-1))


# speedup vs baseline: 1.1996x; 1.0075x over previous
import jax
import jax.numpy as jnp
from jax.experimental import pallas as pl
from jax.experimental.pallas import tpu as pltpu

_N = 4096
_D = 128
_H = 64
_Z = 16
_CAUSAL = 8
_E = 131072
_M = 2
_SIZE_COEF = 1e-4
_KL_COEF = 1.0
_VGAE_COEF = 1.0

_RT = 128
_G = _N // _RT


_CT = 512


def _combine(s, t, deg, W, relu_cols):
    n, k = s.shape
    f = W.shape[1]

    def body(s_ref, t_ref, d_ref, w_ref, o_ref):
        w = w_ref[...]
        sw = jnp.dot(s_ref[...], w, preferred_element_type=jnp.float32)
        tw = jnp.dot(t_ref[...], w, preferred_element_type=jnp.float32)
        o = sw / d_ref[...] + tw
        if relu_cols is None:
            o = jax.nn.relu(o)
        else:
            cols = jax.lax.broadcasted_iota(jnp.int32, (_CT, f), 1)
            o = jnp.where(cols < relu_cols, jax.nn.relu(o), o)
        o_ref[...] = o

    return pl.pallas_call(
        body,
        grid=(n // _CT,),
        in_specs=[
            pl.BlockSpec((_CT, k), lambda i: (i, 0)),
            pl.BlockSpec((_CT, k), lambda i: (i, 0)),
            pl.BlockSpec((_CT, 1), lambda i: (i, 0)),
            pl.BlockSpec((k, f), lambda i: (0, 0)),
        ],
        out_specs=pl.BlockSpec((_CT, f), lambda i: (i, 0)),
        out_shape=jax.ShapeDtypeStruct((n, f), jnp.float32),
    )(s, t, deg, W)


def _sweep_body(z_ref, zt_ref, ct_ref, mu_ref, lv_ref, org_ref, out_ref):
    i = pl.program_id(0)
    z = z_ref[...]
    s = jnp.dot(z, zt_ref[...], preferred_element_type=jnp.float32)
    r = jax.nn.sigmoid(s)
    log_neg = jnp.log(1.0 - r + 1e-8)
    bce_all = jnp.sum(log_neg)
    rows = jax.lax.broadcasted_iota(jnp.int32, (_RT, _N), 0) + i * _RT
    cols = jax.lax.broadcasted_iota(jnp.int32, (_RT, _N), 1)
    on = jnp.logical_or(org_ref[...] > 0.0, rows == cols)
    corr = jnp.sum(jnp.where(on, jnp.log(r + 1e-8) - log_neg, 0.0))
    c = z[:, :_CAUSAL]
    sc = jnp.dot(c, ct_ref[...], preferred_element_type=jnp.float32)
    size_sum = jnp.sum(jax.nn.sigmoid(sc))
    muv = mu_ref[...]
    lvv = lv_ref[...]
    klz_part = jnp.sum(1.0 + lvv - muv * muv - jnp.exp(lvv))
    row = jnp.concatenate([
        jnp.reshape(bce_all + corr, (1,)),
        jnp.reshape(size_sum, (1,)),
        jnp.reshape(klz_part, (1,)),
        jnp.zeros((125,), jnp.float32),
    ])[None, :]
    out_ref[pl.ds(i, 1), :] = row


def _decoder_losses(all_z, mu, logvar, org_mask):
    zt = all_z.T
    ct = all_z[:, :_CAUSAL].T
    out = pl.pallas_call(
        _sweep_body,
        grid=(_G,),
        in_specs=[
            pl.BlockSpec((_RT, _Z), lambda i: (i, 0)),
            pl.BlockSpec((_Z, _N), lambda i: (0, 0)),
            pl.BlockSpec((_CAUSAL, _N), lambda i: (0, 0)),
            pl.BlockSpec((_RT, _Z), lambda i: (i, 0)),
            pl.BlockSpec((_RT, _Z), lambda i: (i, 0)),
            pl.BlockSpec((_RT, _N), lambda i: (i, 0)),
        ],
        out_specs=pl.BlockSpec((_G, 128), lambda i: (0, 0)),
        out_shape=jax.ShapeDtypeStruct((_G, 128), jnp.float32),
    )(all_z, zt, ct, mu, logvar, org_mask)
    sums = jnp.sum(out, axis=0)
    bce = -sums[0] / (_N * _N)
    size_loss = _SIZE_COEF * sums[1] / (_N * _N)
    klz = (-0.5 / _N) * sums[2]
    return bce, size_loss, klz


def kernel(x, edge_index, eps, W1, W_mu, W_lv, Wc1, Wc2, Wout):
    src = edge_index[0]
    dst = edge_index[1]

    x_aug = jnp.concatenate([x, jnp.ones((_N, 1), jnp.float32)], axis=1)
    g_xa = jnp.take(x_aug, src, axis=0)
    s_xa = jax.ops.segment_sum(g_xa, dst, num_segments=_N)
    s_x = s_xa[:, :_D]
    deg = jnp.clip(s_xa[:, _D:], 1.0)

    hh = _combine(s_x, x, deg, jnp.concatenate([Wc1, W1], axis=1), None)
    s_cat = jax.ops.segment_sum(jnp.take(hh, src, axis=0), dst,
                                num_segments=_N)
    w_top = jnp.concatenate([Wc2, jnp.zeros((_H, 2 * _Z), jnp.float32)], axis=1)
    w_bot = jnp.concatenate([jnp.zeros((_H, _H), jnp.float32), W_mu, W_lv], axis=1)
    cat2 = _combine(s_cat, hh, deg, jnp.concatenate([w_top, w_bot], axis=0), _H)
    h2o = cat2[:, :_H]
    mu = cat2[:, _H:_H + _Z]
    logvar = cat2[:, _H + _Z:]
    orig_logits = jnp.mean(h2o, axis=0, keepdims=True) @ Wout
    all_z = mu + jnp.exp(0.5 * logvar) * eps
    caul_z = all_z[:, :_CAUSAL]

    c_both = jnp.take(caul_z, jnp.concatenate([src, dst]), axis=0)
    edge_attn = jax.nn.sigmoid(
        jnp.sum(c_both[:_E] * c_both[_E:], axis=1, keepdims=True))

    s_xw = jax.ops.segment_sum(g_xa[:, :_D] * edge_attn, dst,
                               num_segments=_N)
    h1m = _combine(s_xw, x, deg, Wc1, None)
    s_m = jax.ops.segment_sum(jnp.take(h1m, src, axis=0) * edge_attn, dst,
                              num_segments=_N)
    h2m = _combine(s_m, h1m, deg, Wc2, None)
    masked_logits = jnp.mean(h2m, axis=0, keepdims=True) @ Wout

    org_mask = jnp.zeros((_N, _N), jnp.float32).at[
        jnp.concatenate([src, dst]), jnp.concatenate([dst, src])].add(1.0)

    bce, size_loss, klz = _decoder_losses(all_z, mu, logvar, org_mask)

    logp = jax.nn.log_softmax(masked_logits, axis=1)
    p = jax.nn.softmax(orig_logits, axis=1)
    kl_loss = _KL_COEF * jnp.sum(p * (jnp.log(p + 1e-12) - logp)) / masked_logits.shape[0]
    vgae_loss = _VGAE_COEF * (bce + klz)
    loss = size_loss + kl_loss + vgae_loss
    return (loss, orig_logits, edge_attn.reshape(-1))
